# baseline (speedup 1.0000x reference)
import functools

import jax
import jax.numpy as jnp
from jax import lax
from jax.experimental import pallas as pl
from jax.experimental.pallas import tpu as pltpu
from jax.experimental.pallas import tpu_sc as plsc

VOCAB = 1000000
EMBED = 64
OUT = 2
SEQ = 200
BATCH = 4096

NUM_CORES = 2
NUM_SUBCORES = 16
NUM_WORKERS = NUM_CORES * NUM_SUBCORES
B_PER_W = BATCH // NUM_WORKERS
LANES = 16
TOKENS_PER_W = SEQ * B_PER_W
KCHUNK = 1024
NCHUNK = TOKENS_PER_W // KCHUNK
VCHUNK = 65536
NBUF = 5


def _tc_project(embed_table, W, b):
  et = embed_table.T

  def body(w_ref, b_ref, et_ref, o_ref):
    p = lax.dot_general(w_ref[...], et_ref[...], (((1,), (0,)), ((), ())),
                        preferred_element_type=jnp.float32)
    p = (p + b_ref[...]) * (1.0 / SEQ)
    u = lax.bitcast_convert_type(
        p.astype(jnp.bfloat16), jnp.uint16).astype(jnp.uint32)
    packed = lax.shift_left(u[1], jnp.uint32(16)) | u[0]
    o_ref[...] = lax.bitcast_convert_type(packed, jnp.float32)

  return pl.pallas_call(
      body,
      grid=(pl.cdiv(VOCAB, VCHUNK),),
      in_specs=[
          pl.BlockSpec((OUT, EMBED), lambda i: (0, 0)),
          pl.BlockSpec((OUT, 1), lambda i: (0, 0)),
          pl.BlockSpec((EMBED, VCHUNK), lambda i: (0, i)),
      ],
      out_specs=pl.BlockSpec((VCHUNK,), lambda i: (i,)),
      out_shape=jax.ShapeDtypeStruct((VOCAB,), jnp.float32),
  )(W, b.reshape(OUT, 1), et)


def _sc_gather_sum(text, packed):
  mesh = plsc.VectorSubcoreMesh(
      core_axis_name="c", subcore_axis_name="s",
      num_cores=NUM_CORES, num_subcores=NUM_SUBCORES)

  @functools.partial(
      pl.kernel,
      out_type=(
          jax.ShapeDtypeStruct((BATCH,), jnp.float32),
          jax.ShapeDtypeStruct((BATCH,), jnp.float32),
      ),
      mesh=mesh,
      compiler_params=pltpu.CompilerParams(use_tc_tiling_on_sc=False),
      scratch_types=[
          pltpu.VMEM((TOKENS_PER_W,), jnp.int32),
          pltpu.VMEM((NBUF, KCHUNK), jnp.float32),
          pltpu.VMEM((B_PER_W,), jnp.float32),
          pltpu.VMEM((B_PER_W,), jnp.float32),
          pltpu.SemaphoreType.DMA,
          pltpu.SemaphoreType.DMA,
      ],
  )
  def k(text_hbm, p_hbm, out0_hbm, out1_hbm,
        idx_v, buf_v, out0_v, out1_v, sem, fsem):
    wid = lax.axis_index("s") * NUM_CORES + lax.axis_index("c")
    base = wid * B_PER_W

    @pl.loop(0, SEQ)
    def fill(s):
      pltpu.make_async_copy(
          text_hbm.at[s, pl.ds(base, B_PER_W)],
          idx_v.at[pl.ds(s * B_PER_W, B_PER_W)], fsem).start()

    @pl.loop(0, SEQ)
    def fill_drain(s):
      pltpu.make_async_copy(
          text_hbm.at[0, pl.ds(base, B_PER_W)],
          idx_v.at[pl.ds(0, B_PER_W)], fsem).wait()

    def fire(c, slot):
      chunk = idx_v.at[pl.ds(c * KCHUNK, KCHUNK)]
      pltpu.make_async_copy(p_hbm.at[chunk], buf_v.at[slot], sem).start()

    def wait_one(slot):
      chunk0 = idx_v.at[pl.ds(0, KCHUNK)]
      pltpu.make_async_copy(p_hbm.at[chunk0], buf_v.at[slot], sem).wait()

    for slot in range(NBUF):
      fire(slot, slot)

    zeros = jnp.zeros((LANES,), jnp.float32)
    nv = B_PER_W // LANES
    subs = KCHUNK // B_PER_W
    himask = jnp.full((LANES,), 0xFFFF0000, jnp.uint32)
    sh16 = jnp.full((LANES,), 16, jnp.uint32)

    @pl.loop(0, NCHUNK, step=NBUF,
             init_carry=tuple(zeros for _ in range(2 * nv)))
    def outer(g, accs):
      for slot in range(NBUF):
        c = g + slot
        wait_one(slot)
        for sub in range(subs):
          new = list(accs)
          for j in range(nv):
            x = buf_v[slot, pl.ds(sub * B_PER_W + j * LANES, LANES)]
            xu = lax.bitcast_convert_type(x, jnp.uint32)
            p0 = lax.bitcast_convert_type(
                lax.shift_left(xu, sh16), jnp.float32)
            p1 = lax.bitcast_convert_type(xu & himask, jnp.float32)
            new[j] = new[j] + p0
            new[nv + j] = new[nv + j] + p1
          accs = tuple(new)

        @pl.when(c + NBUF < NCHUNK)
        def _():
          fire(c + NBUF, slot)
      return accs

    accs = outer
    for j in range(nv):
      out0_v[pl.ds(j * LANES, LANES)] = accs[j]
      out1_v[pl.ds(j * LANES, LANES)] = accs[nv + j]
    pltpu.sync_copy(out0_v, out0_hbm.at[pl.ds(base, B_PER_W)])
    pltpu.sync_copy(out1_v, out1_hbm.at[pl.ds(base, B_PER_W)])

  return k(text, packed)


def kernel(text, embed_table, W, b):
  packed = _tc_project(embed_table, W, b)
  out0, out1 = _sc_gather_sum(text, packed)
  return jnp.stack([out0, out1], axis=1)


# speedup vs baseline: 1.0310x; 1.0310x over previous
import functools

import jax
import jax.numpy as jnp
from jax import lax
from jax.experimental import pallas as pl
from jax.experimental.pallas import tpu as pltpu
from jax.experimental.pallas import tpu_sc as plsc

VOCAB = 1000000
EMBED = 64
OUT = 2
SEQ = 200
BATCH = 4096

NUM_CORES = 2
NUM_SUBCORES = 16
NUM_WORKERS = NUM_CORES * NUM_SUBCORES
B_PER_W = BATCH // NUM_WORKERS
LANES = 16
TOKENS_PER_W = SEQ * B_PER_W
KCHUNK = 1024
NCHUNK = TOKENS_PER_W // KCHUNK
VCHUNK = 32768
NBUF = 5


def _tc_project(embed_table, W, b):
  et = embed_table.T

  def body(w_ref, b_ref, et_ref, o_ref):
    p = lax.dot_general(w_ref[...], et_ref[...], (((1,), (0,)), ((), ())),
                        preferred_element_type=jnp.float32)
    p = (p + b_ref[...]) * (1.0 / SEQ)
    u = lax.bitcast_convert_type(
        p.astype(jnp.bfloat16), jnp.uint16).astype(jnp.uint32)
    packed = lax.shift_left(u[1], jnp.uint32(16)) | u[0]
    o_ref[...] = lax.bitcast_convert_type(packed, jnp.float32)

  return pl.pallas_call(
      body,
      grid=(pl.cdiv(VOCAB, VCHUNK),),
      in_specs=[
          pl.BlockSpec((OUT, EMBED), lambda i: (0, 0)),
          pl.BlockSpec((OUT, 1), lambda i: (0, 0)),
          pl.BlockSpec((EMBED, VCHUNK), lambda i: (0, i)),
      ],
      out_specs=pl.BlockSpec((VCHUNK,), lambda i: (i,)),
      out_shape=jax.ShapeDtypeStruct((VOCAB,), jnp.float32),
  )(W, b.reshape(OUT, 1), et)


def _sc_gather_sum(text, packed):
  mesh = plsc.VectorSubcoreMesh(
      core_axis_name="c", subcore_axis_name="s",
      num_cores=NUM_CORES, num_subcores=NUM_SUBCORES)

  @functools.partial(
      pl.kernel,
      out_type=(
          jax.ShapeDtypeStruct((BATCH,), jnp.float32),
          jax.ShapeDtypeStruct((BATCH,), jnp.float32),
      ),
      mesh=mesh,
      compiler_params=pltpu.CompilerParams(use_tc_tiling_on_sc=False),
      scratch_types=[
          pltpu.VMEM((TOKENS_PER_W,), jnp.int32),
          pltpu.VMEM((NBUF, KCHUNK), jnp.float32),
          pltpu.VMEM((B_PER_W,), jnp.float32),
          pltpu.VMEM((B_PER_W,), jnp.float32),
          pltpu.SemaphoreType.DMA,
          pltpu.SemaphoreType.DMA,
      ],
  )
  def k(text_hbm, p_hbm, out0_hbm, out1_hbm,
        idx_v, buf_v, out0_v, out1_v, sem, fsem):
    wid = lax.axis_index("s") * NUM_CORES + lax.axis_index("c")
    base = wid * B_PER_W

    subs = KCHUNK // B_PER_W

    def fill_chunk(c):
      for r in range(subs):
        s = c * subs + r
        pltpu.make_async_copy(
            text_hbm.at[s, pl.ds(base, B_PER_W)],
            idx_v.at[pl.ds(s * B_PER_W, B_PER_W)], fsem).start()

    def drain_chunk_fills():
      for _ in range(subs):
        pltpu.make_async_copy(
            text_hbm.at[0, pl.ds(base, B_PER_W)],
            idx_v.at[pl.ds(0, B_PER_W)], fsem).wait()

    def fire(c, slot):
      chunk = idx_v.at[pl.ds(c * KCHUNK, KCHUNK)]
      pltpu.make_async_copy(p_hbm.at[chunk], buf_v.at[slot], sem).start()

    def wait_one(slot):
      chunk0 = idx_v.at[pl.ds(0, KCHUNK)]
      pltpu.make_async_copy(p_hbm.at[chunk0], buf_v.at[slot], sem).wait()

    fill_ahead = min(2 * NBUF, NCHUNK)
    for c in range(fill_ahead):
      fill_chunk(c)
    for slot in range(NBUF):
      drain_chunk_fills()
      fire(slot, slot)

    zeros = jnp.zeros((LANES,), jnp.float32)
    nv = B_PER_W // LANES
    himask = jnp.full((LANES,), 0xFFFF0000, jnp.uint32)
    sh16 = jnp.full((LANES,), 16, jnp.uint32)

    @pl.loop(0, NCHUNK, step=NBUF,
             init_carry=tuple(zeros for _ in range(2 * nv)))
    def outer(g, accs):
      for slot in range(NBUF):
        c = g + slot
        wait_one(slot)
        for sub in range(subs):
          new = list(accs)
          for j in range(nv):
            x = buf_v[slot, pl.ds(sub * B_PER_W + j * LANES, LANES)]
            xu = lax.bitcast_convert_type(x, jnp.uint32)
            p0 = lax.bitcast_convert_type(
                lax.shift_left(xu, sh16), jnp.float32)
            p1 = lax.bitcast_convert_type(xu & himask, jnp.float32)
            new[j] = new[j] + p0
            new[nv + j] = new[nv + j] + p1
          accs = tuple(new)

        @pl.when(c + 2 * NBUF < NCHUNK)
        def _():
          fill_chunk(c + 2 * NBUF)

        @pl.when(c + NBUF < NCHUNK)
        def _():
          drain_chunk_fills()
          fire(c + NBUF, slot)
      return accs

    accs = outer
    for j in range(nv):
      out0_v[pl.ds(j * LANES, LANES)] = accs[j]
      out1_v[pl.ds(j * LANES, LANES)] = accs[nv + j]
    pltpu.sync_copy(out0_v, out0_hbm.at[pl.ds(base, B_PER_W)])
    pltpu.sync_copy(out1_v, out1_hbm.at[pl.ds(base, B_PER_W)])

  return k(text, packed)


def kernel(text, embed_table, W, b):
  packed = _tc_project(embed_table, W, b)
  out0, out1 = _sc_gather_sum(text, packed)
  return jnp.stack([out0, out1], axis=1)
